# baseline (speedup 1.0000x reference)
import functools

import jax
import jax.numpy as jnp
from jax import lax
from jax.experimental import pallas as pl
from jax.experimental.pallas import tpu as pltpu, tpu_sc as plsc

N = 32768
NRAYS = 16
NC = 2
NSUB = 16
CH = N // NSUB
SH = 11
L = 16
LPC = CH // L
EPS = 1e-6
_LN2 = 0.6931471805599453
_SQRT2 = 1.4142135623730951
_U1 = 8
_U2 = 8

_mesh = plsc.VectorSubcoreMesh(
    core_axis_name="c", subcore_axis_name="s", num_cores=NC, num_subcores=NSUB
)


def _ln16(x):
    bits = lax.bitcast_convert_type(x, jnp.int32)
    e = lax.shift_right_arithmetic(bits, 23) - 127
    m = lax.bitcast_convert_type(
        (bits & 0x007FFFFF) | 0x3F800000, jnp.float32
    )
    big = m > _SQRT2
    m = jnp.where(big, m * 0.5, m)
    e = jnp.where(big, e + 1, e)
    s = (m - 1.0) / (m + 1.0)
    z = s * s
    p = 1.0 + z * (
        (1.0 / 3.0)
        + z * ((1.0 / 5.0) + z * ((1.0 / 7.0) + z * ((1.0 / 9.0) + z * (1.0 / 11.0))))
    )
    return 2.0 * s * p + e.astype(jnp.float32) * _LN2


def _dyn_gather(x, idx):
    dnums = lax.GatherDimensionNumbers(
        offset_dims=(), collapsed_slice_dims=(0,), start_index_map=(0,)
    )
    return lax.gather(
        x, idx[:, None], dnums, (1,),
        mode=lax.GatherScatterMode.PROMISE_IN_BOUNDS,
    )


def _body(
    cu_hbm, alpha_hbm, trans_hbm, bg_hbm,
    alpha_v, tr_v, dseg_v, out_v, cu_v, pub_v, all_v, bg_v, shared_v,
):
    s = lax.axis_index("s")
    c = lax.axis_index("c")
    base = s * CH
    pltpu.sync_copy(alpha_hbm.at[pl.ds(base, CH)], alpha_v)
    pltpu.sync_copy(cu_hbm.at[pl.ds(0, L)], cu_v)
    iot = lax.iota(jnp.int32, L)
    idxb = iot * LPC

    def s1(t0, acc):
        for dt in range(_U1):
            t = t0 * _U1 + dt
            a = plsc.load_gather(alpha_v, [idxb + t])
            x = jnp.minimum(jnp.maximum(1.0 - a, EPS), 1.0)
            l = _ln16(x)
            tr_v[pl.ds(t * L, L)] = acc
            dseg_v[pl.ds(t * L, L)] = jnp.zeros((L,), jnp.int32)
            acc = acc + l
        return acc

    lane_tot = lax.fori_loop(0, LPC // _U1, s1, jnp.zeros((L,), jnp.float32))
    lane_cum = plsc.cumsum(lane_tot)
    lane_ex = lane_cum - lane_tot
    total = lane_cum[L - 1]

    S = cu_v[...]
    in_s = (S >= base) & (S < base + CH)
    q = jnp.clip(S - base, 0, CH - 1)
    lane_of = lax.shift_right_arithmetic(q, 7)
    t_of = q & (LPC - 1)
    loc = plsc.load_gather(tr_v, [t_of * L + lane_of]) + _dyn_gather(lane_ex, lane_of)
    pub_v[0, :] = jnp.where(in_s, loc, 0.0)
    pub_v[1, :] = jnp.full((L,), total, jnp.float32)
    pltpu.sync_copy(pub_v, shared_v.at[s])
    plsc.subcore_barrier()
    pltpu.sync_copy(shared_v, all_v)

    sv = jnp.zeros((L,), jnp.float32)
    for w in range(NSUB):
        sv = sv + all_v[w, 0, :]

    c_sv = lax.shift_right_arithmetic(S, SH)
    idx1 = jnp.minimum(iot + 1, L - 1)
    last = iot == L - 1
    c_ev = jnp.where(last, NSUB, _dyn_gather(c_sv, idx1))
    ev = jnp.where(last, 0.0, _dyn_gather(sv, idx1))

    D = jnp.zeros((L,), jnp.float32)
    BD = jnp.zeros((L,), jnp.float32)
    for w in range(NSUB):
        tot_w = all_v[w, 1, :][0]
        m_ge = c_sv <= w
        D = D + jnp.where(m_ge & (w < s), tot_w, 0.0)
        BD = BD + jnp.where(m_ge & (w < c_ev), tot_w, 0.0)
    R = D - sv

    bg_v[...] = jnp.exp((ev - sv) + BD)

    @pl.when((s == 0) & (c == 0))
    def _():
        pltpu.sync_copy(bg_v, bg_hbm)

    plsc.addupdate_scatter(
        dseg_v, [q], jnp.ones((L,), jnp.int32), mask=in_s & (iot >= 1)
    )
    p0 = base + idxb
    seg0 = jnp.zeros((L,), jnp.int32)
    for j in range(1, L):
        seg0 = seg0 + (S[j] < p0).astype(jnp.int32)

    def s2(t0, seg_acc):
        for dt in range(_U2):
            t = t0 * _U2 + dt
            idx = idxb + t
            seg_acc = seg_acc + plsc.load_gather(dseg_v, [idx])
            rsel = _dyn_gather(R, seg_acc)
            ex = tr_v[pl.ds(t * L, L)]
            tv = jnp.exp((ex + rsel) + lane_ex)
            plsc.store_scatter(out_v, [idx], tv)
        return seg_acc

    lax.fori_loop(0, LPC // _U2, s2, seg0)
    pltpu.sync_copy(out_v, trans_hbm.at[pl.ds(base, CH)])


_kernel = functools.partial(
    pl.kernel,
    out_type=(
        jax.ShapeDtypeStruct((N,), jnp.float32),
        jax.ShapeDtypeStruct((NRAYS,), jnp.float32),
    ),
    mesh=_mesh,
    scratch_types=[
        pltpu.VMEM((CH,), jnp.float32),
        pltpu.VMEM((CH,), jnp.float32),
        pltpu.VMEM((CH,), jnp.int32),
        pltpu.VMEM((CH,), jnp.float32),
        pltpu.VMEM((L,), jnp.int32),
        pltpu.VMEM((2, L), jnp.float32),
        pltpu.VMEM((NSUB, 2, L), jnp.float32),
        pltpu.VMEM((L,), jnp.float32),
        pltpu.VMEM_SHARED((NSUB, 2, L), jnp.float32),
    ],
    compiler_params=pltpu.CompilerParams(needs_layout_passes=False),
)(_body)


def kernel(cu_seqlens, alpha):
    transmittance, bg_transmittance = _kernel(cu_seqlens, alpha)
    return transmittance, bg_transmittance


# speedup vs baseline: 1.1929x; 1.1929x over previous
import functools

import jax
import jax.numpy as jnp
from jax import lax
from jax.experimental import pallas as pl
from jax.experimental.pallas import tpu as pltpu, tpu_sc as plsc

N = 32768
NRAYS = 16
NC = 2
NSUB = 16
CH = N // NSUB
SH = 11
L = 16
LPC = CH // L
EPS = 1e-6
_LN2 = 0.6931471805599453
_SQRT2 = 1.4142135623730951
_U1 = 8
_U2 = 8

_mesh = plsc.VectorSubcoreMesh(
    core_axis_name="c", subcore_axis_name="s", num_cores=NC, num_subcores=NSUB
)


def _ln16(x):
    bits = lax.bitcast_convert_type(x, jnp.int32)
    e = lax.shift_right_arithmetic(bits, 23) - 127
    m = lax.bitcast_convert_type(
        (bits & 0x007FFFFF) | 0x3F800000, jnp.float32
    )
    big = m > _SQRT2
    m = jnp.where(big, m * 0.5, m)
    e = jnp.where(big, e + 1, e)
    s = (m - 1.0) / (m + 1.0)
    z = s * s
    p = 1.0 + z * (
        (1.0 / 3.0)
        + z * ((1.0 / 5.0) + z * ((1.0 / 7.0) + z * ((1.0 / 9.0) + z * (1.0 / 11.0))))
    )
    return 2.0 * s * p + e.astype(jnp.float32) * _LN2


def _dyn_gather(x, idx):
    dnums = lax.GatherDimensionNumbers(
        offset_dims=(), collapsed_slice_dims=(0,), start_index_map=(0,)
    )
    return lax.gather(
        x, idx[:, None], dnums, (1,),
        mode=lax.GatherScatterMode.PROMISE_IN_BOUNDS,
    )


def _body(
    cu_hbm, alpha_hbm, trans_hbm, bg_hbm,
    alpha_v, tr_v, dseg_v, out_v, cu_v, pub_v, all_v, bg_v, shared_v,
):
    s = lax.axis_index("s")
    c = lax.axis_index("c")
    base = s * CH
    pltpu.sync_copy(alpha_hbm.at[pl.ds(base, CH)], alpha_v)
    pltpu.sync_copy(cu_hbm.at[pl.ds(0, L)], cu_v)
    iot = lax.iota(jnp.int32, L)
    idxb = iot * LPC

    @plsc.parallel_loop(0, LPC, unroll=_U1, carry=jnp.zeros((L,), jnp.float32))
    def s1(t, acc):
        a = plsc.load_gather(alpha_v, [idxb + t])
        x = jnp.minimum(jnp.maximum(1.0 - a, EPS), 1.0)
        l = _ln16(x)
        tr_v[pl.ds(t * L, L)] = acc
        dseg_v[pl.ds(t * L, L)] = jnp.zeros((L,), jnp.int32)
        return acc + l

    lane_tot = s1
    lane_cum = plsc.cumsum(lane_tot)
    lane_ex = lane_cum - lane_tot
    total = lane_cum[L - 1]

    S = cu_v[...]
    in_s = (S >= base) & (S < base + CH)
    q = jnp.clip(S - base, 0, CH - 1)
    lane_of = lax.shift_right_arithmetic(q, 7)
    t_of = q & (LPC - 1)
    loc = plsc.load_gather(tr_v, [t_of * L + lane_of]) + _dyn_gather(lane_ex, lane_of)
    pub_v[0, :] = jnp.where(in_s, loc, 0.0)
    pub_v[1, :] = jnp.full((L,), total, jnp.float32)
    pltpu.sync_copy(pub_v, shared_v.at[s])
    plsc.subcore_barrier()
    pltpu.sync_copy(shared_v, all_v)

    sv = jnp.zeros((L,), jnp.float32)
    for w in range(NSUB):
        sv = sv + all_v[w, 0, :]

    c_sv = lax.shift_right_arithmetic(S, SH)
    idx1 = jnp.minimum(iot + 1, L - 1)
    last = iot == L - 1
    c_ev = jnp.where(last, NSUB, _dyn_gather(c_sv, idx1))
    ev = jnp.where(last, 0.0, _dyn_gather(sv, idx1))

    D = jnp.zeros((L,), jnp.float32)
    BD = jnp.zeros((L,), jnp.float32)
    for w in range(NSUB):
        tot_w = all_v[w, 1, :][0]
        m_ge = c_sv <= w
        D = D + jnp.where(m_ge & (w < s), tot_w, 0.0)
        BD = BD + jnp.where(m_ge & (w < c_ev), tot_w, 0.0)
    R = D - sv

    bg_v[...] = jnp.exp((ev - sv) + BD)

    @pl.when((s == 0) & (c == 0))
    def _():
        pltpu.sync_copy(bg_v, bg_hbm)

    plsc.addupdate_scatter(
        dseg_v, [q], jnp.ones((L,), jnp.int32), mask=in_s & (iot >= 1)
    )
    p0 = base + idxb
    seg0 = jnp.zeros((L,), jnp.int32)
    for j in range(1, L):
        seg0 = seg0 + (S[j] < p0).astype(jnp.int32)

    @plsc.parallel_loop(0, LPC, unroll=_U2, carry=seg0)
    def s2(t, seg_acc):
        idx = idxb + t
        seg_acc = seg_acc + plsc.load_gather(dseg_v, [idx])
        rsel = _dyn_gather(R, seg_acc)
        ex = tr_v[pl.ds(t * L, L)]
        tv = jnp.exp((ex + rsel) + lane_ex)
        plsc.store_scatter(out_v, [idx], tv)
        return seg_acc

    del s2
    pltpu.sync_copy(out_v, trans_hbm.at[pl.ds(base, CH)])


_kernel = functools.partial(
    pl.kernel,
    out_type=(
        jax.ShapeDtypeStruct((N,), jnp.float32),
        jax.ShapeDtypeStruct((NRAYS,), jnp.float32),
    ),
    mesh=_mesh,
    scratch_types=[
        pltpu.VMEM((CH,), jnp.float32),
        pltpu.VMEM((CH,), jnp.float32),
        pltpu.VMEM((CH,), jnp.int32),
        pltpu.VMEM((CH,), jnp.float32),
        pltpu.VMEM((L,), jnp.int32),
        pltpu.VMEM((2, L), jnp.float32),
        pltpu.VMEM((NSUB, 2, L), jnp.float32),
        pltpu.VMEM((L,), jnp.float32),
        pltpu.VMEM_SHARED((NSUB, 2, L), jnp.float32),
    ],
    compiler_params=pltpu.CompilerParams(needs_layout_passes=False),
)(_body)


def kernel(cu_seqlens, alpha):
    transmittance, bg_transmittance = _kernel(cu_seqlens, alpha)
    return transmittance, bg_transmittance
